# baseline (speedup 1.0000x reference)
import functools

import jax
import jax.numpy as jnp
from jax import lax
from jax.experimental import pallas as pl
from jax.experimental.pallas import tpu as pltpu
from jax.experimental.pallas import tpu_sc as plsc

_B, _N, _M, _K = 8, 4096, 1024, 16
_C_IN, _C_OUT = 64, 128
_CF = _C_IN + 3
_EPS = 1e-5


def _fps_body(xt_ref, xyz_ref, nxyz_ref):
    x = xt_ref[0]
    y = xt_ref[1]
    z = xt_ref[2]
    xyz = xyz_ref[...]
    iota_n = lax.broadcasted_iota(jnp.int32, (_B, _N), 1)
    iota_m = lax.broadcasted_iota(jnp.int32, (_B, _M), 1)

    def cur_coords(last):
        oh = (iota_n == last).astype(jnp.float32)
        cur = lax.dot_general(oh, xyz, (((1,), (1,)), ((0,), (0,))),
                              preferred_element_type=jnp.float32)
        return cur[:, 0:1], cur[:, 1:2], cur[:, 2:3]

    d0 = jnp.full((_B, _N), 1e10, dtype=jnp.float32)
    last0 = jnp.zeros((_B, 1), jnp.int32)
    zm = jnp.zeros((_B, _M), jnp.float32)

    def body(i, st):
        d, last, nx, ny, nz = st
        cx, cy, cz = cur_coords(last)
        selm = iota_m == (i - 1)
        nx = jnp.where(selm, cx, nx)
        ny = jnp.where(selm, cy, ny)
        nz = jnp.where(selm, cz, nz)
        dist = (x - cx) ** 2 + (y - cy) ** 2 + (z - cz) ** 2
        d = jnp.minimum(d, dist)
        mx = jnp.max(d, axis=1, keepdims=True)
        nxt = jnp.min(jnp.where(d == mx, iota_n, _N), axis=1, keepdims=True)
        nxt = nxt.astype(jnp.int32)
        return (d, nxt, nx, ny, nz)

    d, last, nx, ny, nz = lax.fori_loop(1, _M, body, (d0, last0, zm, zm, zm))
    cx, cy, cz = cur_coords(last)
    selm = iota_m == (_M - 1)
    nxyz_ref[0] = jnp.where(selm, cx, nx)
    nxyz_ref[1] = jnp.where(selm, cy, ny)
    nxyz_ref[2] = jnp.where(selm, cz, nz)


def _fps(xt, xyz):
    return pl.pallas_call(
        _fps_body,
        out_shape=jax.ShapeDtypeStruct((3, _B, _M), jnp.float32),
    )(xt, xyz)


_MB2 = 256


def _knn_body(xt_ref, nx_ref, idx_ref):
    xp = xt_ref[0]
    t = nx_ref[0]
    p2 = jnp.sum(xp * xp, axis=0, keepdims=True)
    t2 = jnp.sum(t * t, axis=0)
    dot = lax.dot_general(t, xp, (((0,), (0,)), ((), ())),
                          preferred_element_type=jnp.float32)
    d2 = t2[:, None] + p2 - 2.0 * dot
    iota_n = lax.broadcasted_iota(jnp.int32, (_MB2, _N), 1)
    for kk in range(_K):
        mn = jnp.min(d2, axis=1, keepdims=True)
        eq = d2 == mn
        am = jnp.min(jnp.where(eq, iota_n, _N), axis=1, keepdims=True)
        am = am.astype(jnp.int32)
        idx_ref[0, :, kk:kk + 1] = am
        d2 = jnp.where(eq, jnp.inf, d2)


def _knn(xt, nxyz):
    return pl.pallas_call(
        _knn_body,
        grid=(_B, _M // _MB2),
        in_specs=[
            pl.BlockSpec((1, 3, _N), lambda b, mb: (b, 0, 0)),
            pl.BlockSpec((1, 3, _MB2), lambda b, mb: (b, 0, mb)),
        ],
        out_specs=pl.BlockSpec((1, _MB2, _K), lambda b, mb: (b, mb, 0)),
        out_shape=jax.ShapeDtypeStruct((_B, _M, _K), jnp.int32),
    )(xt, nxyz)


def _t2_body(xf_ref, w_ref, beta_ref, t2_ref):
    t2_ref[0] = (
        jnp.dot(xf_ref[0], w_ref[...], preferred_element_type=jnp.float32)
        + beta_ref[...]
    )


def _t2(xf, ws, beta):
    return pl.pallas_call(
        _t2_body,
        grid=(_B,),
        in_specs=[
            pl.BlockSpec((1, _N, _CF), lambda b: (b, 0, 0)),
            pl.BlockSpec((_CF, _C_OUT), lambda b: (0, 0)),
            pl.BlockSpec((1, _C_OUT), lambda b: (0, 0)),
        ],
        out_specs=pl.BlockSpec((1, _N, _C_OUT), lambda b: (b, 0, 0)),
        out_shape=jax.ShapeDtypeStruct((_B, _N, _C_OUT), jnp.float32),
    )(xf, ws, beta)


_NC, _NS, _L = 2, 16, 16
_NW = _NC * _NS
_RPW = (_B * _M) // _NW
_NBUF = 4


def _sc_gather_max_body(t2_hbm, idx_hbm, out_hbm, idx_v, bufs, sems, acc):
    wid = lax.axis_index("s") * _NC + lax.axis_index("c")
    base = wid * _RPW
    boff = (base // _M) * _N
    pltpu.sync_copy(idx_hbm.at[pl.ds(base * _K, _RPW * _K)], idx_v)

    def issue(m, j):
        iv = idx_v[pl.ds(m * _K, _K)] + boff
        pltpu.async_copy(t2_hbm.at[iv], bufs[j], sems[j])

    def wait(j):
        pltpu.make_async_copy(t2_hbm.at[pl.ds(0, _K)], bufs[j], sems[j]).wait()

    def compute(m, j):
        buf = bufs[j]
        for c in range(_C_OUT // _L):
            cs = pl.ds(c * _L, _L)
            mx = buf[0, cs]
            for k in range(1, _K):
                mx = jnp.maximum(mx, buf[k, cs])
            acc[m, cs] = jnp.maximum(mx, 0.0)

    for j in range(_NBUF):
        issue(j, j)

    def body(t, carry):
        for j in range(_NBUF):
            m = t * _NBUF + j
            wait(j)
            compute(m, j)

            @pl.when(m + _NBUF < _RPW)
            def _():
                issue(m + _NBUF, j)

        return carry

    lax.fori_loop(0, _RPW // _NBUF, body, 0)
    pltpu.sync_copy(acc, out_hbm.at[pl.ds(base, _RPW)])


def _sc_gather_max(t2_flat, idx_flat):
    mesh = plsc.VectorSubcoreMesh(core_axis_name="c", subcore_axis_name="s")
    f = functools.partial(
        pl.kernel,
        out_type=jax.ShapeDtypeStruct((_B * _M, _C_OUT), jnp.float32),
        mesh=mesh,
        scratch_types=[
            pltpu.VMEM((_RPW * _K,), jnp.int32),
            [pltpu.VMEM((_K, _C_OUT), jnp.float32) for _ in range(_NBUF)],
            [pltpu.SemaphoreType.DMA for _ in range(_NBUF)],
            pltpu.VMEM((_RPW, _C_OUT), jnp.float32),
        ],
    )(_sc_gather_max_body)
    return f(t2_flat, idx_flat)


def kernel(points_xyz, features, W, gamma, beta):
    xt = jnp.transpose(points_xyz, (2, 0, 1))
    nxyz = _fps(xt, points_xyz)
    new_xyz = jnp.transpose(nxyz, (1, 2, 0))

    idx = _knn(jnp.transpose(xt, (1, 0, 2)), jnp.transpose(nxyz, (1, 0, 2)))

    scale = gamma / jnp.sqrt(1.0 + _EPS)
    ws = (W * scale[:, None]).T
    xf = jnp.concatenate(
        [points_xyz, jnp.transpose(features, (0, 2, 1))], axis=-1)
    t2 = _t2(xf, ws, beta[None, :])
    pooled = _sc_gather_max(
        t2.reshape(_B * _N, _C_OUT), idx.reshape(_B * _M * _K))
    pooled = pooled.reshape(_B, _M, _C_OUT)
    return (new_xyz, jnp.transpose(pooled, (0, 2, 1)))


# speedup vs baseline: 2.1501x; 2.1501x over previous
import functools

import jax
import jax.numpy as jnp
from jax import lax
from jax.experimental import pallas as pl
from jax.experimental.pallas import tpu as pltpu
from jax.experimental.pallas import tpu_sc as plsc

_B, _N, _M, _K = 8, 4096, 1024, 16
_C_IN, _C_OUT = 64, 128
_CF = _C_IN + 3
_EPS = 1e-5


def _fps_body(xt_ref, nxyz_ref):
    x = xt_ref[0]
    y = xt_ref[1]
    z = xt_ref[2]
    iota_n = lax.broadcasted_iota(jnp.int32, (_B, _N), 1)
    iota_m = lax.broadcasted_iota(jnp.int32, (_B, _M), 1)

    def cur_coords(last):
        sel = iota_n == last
        cx = jnp.sum(jnp.where(sel, x, 0.0), axis=1, keepdims=True)
        cy = jnp.sum(jnp.where(sel, y, 0.0), axis=1, keepdims=True)
        cz = jnp.sum(jnp.where(sel, z, 0.0), axis=1, keepdims=True)
        return cx, cy, cz

    d0 = jnp.full((_B, _N), 1e10, dtype=jnp.float32)
    last0 = jnp.zeros((_B, 1), jnp.int32)
    zm = jnp.zeros((_B, _M), jnp.float32)

    def body(i, st):
        d, last, nx, ny, nz = st
        cx, cy, cz = cur_coords(last)
        selm = iota_m == (i - 1)
        nx = jnp.where(selm, cx, nx)
        ny = jnp.where(selm, cy, ny)
        nz = jnp.where(selm, cz, nz)
        dist = (x - cx) ** 2 + (y - cy) ** 2 + (z - cz) ** 2
        d = jnp.minimum(d, dist)
        mx = jnp.max(d, axis=1, keepdims=True)
        nxt = jnp.min(jnp.where(d == mx, iota_n, _N), axis=1, keepdims=True)
        nxt = nxt.astype(jnp.int32)
        return (d, nxt, nx, ny, nz)

    d, last, nx, ny, nz = lax.fori_loop(1, _M, body, (d0, last0, zm, zm, zm))
    cx, cy, cz = cur_coords(last)
    selm = iota_m == (_M - 1)
    nxyz_ref[0] = jnp.where(selm, cx, nx)
    nxyz_ref[1] = jnp.where(selm, cy, ny)
    nxyz_ref[2] = jnp.where(selm, cz, nz)


def _fps(xt):
    return pl.pallas_call(
        _fps_body,
        out_shape=jax.ShapeDtypeStruct((3, _B, _M), jnp.float32),
    )(xt)


_MB2 = 256


def _knn_body(xt_ref, nx_ref, idx_ref):
    xp = xt_ref[0]
    t = nx_ref[0]
    p2 = jnp.sum(xp * xp, axis=0, keepdims=True)
    t2 = jnp.sum(t * t, axis=0)
    dot = lax.dot_general(t, xp, (((0,), (0,)), ((), ())),
                          preferred_element_type=jnp.float32)
    d2 = t2[:, None] + p2 - 2.0 * dot
    iota_n = lax.broadcasted_iota(jnp.int32, (_MB2, _N), 1)
    for kk in range(_K):
        mn = jnp.min(d2, axis=1, keepdims=True)
        eq = d2 == mn
        am = jnp.min(jnp.where(eq, iota_n, _N), axis=1, keepdims=True)
        am = am.astype(jnp.int32)
        idx_ref[0, :, kk:kk + 1] = am
        d2 = jnp.where(eq, jnp.inf, d2)


def _knn(xt, nxyz):
    return pl.pallas_call(
        _knn_body,
        grid=(_B, _M // _MB2),
        in_specs=[
            pl.BlockSpec((1, 3, _N), lambda b, mb: (b, 0, 0)),
            pl.BlockSpec((1, 3, _MB2), lambda b, mb: (b, 0, mb)),
        ],
        out_specs=pl.BlockSpec((1, _MB2, _K), lambda b, mb: (b, mb, 0)),
        out_shape=jax.ShapeDtypeStruct((_B, _M, _K), jnp.int32),
    )(xt, nxyz)


def _t2_body(xf_ref, w_ref, beta_ref, t2_ref):
    t2_ref[0] = (
        jnp.dot(xf_ref[0], w_ref[...], preferred_element_type=jnp.float32)
        + beta_ref[...]
    )


def _t2(xf, ws, beta):
    return pl.pallas_call(
        _t2_body,
        grid=(_B,),
        in_specs=[
            pl.BlockSpec((1, _N, _CF), lambda b: (b, 0, 0)),
            pl.BlockSpec((_CF, _C_OUT), lambda b: (0, 0)),
            pl.BlockSpec((1, _C_OUT), lambda b: (0, 0)),
        ],
        out_specs=pl.BlockSpec((1, _N, _C_OUT), lambda b: (b, 0, 0)),
        out_shape=jax.ShapeDtypeStruct((_B, _N, _C_OUT), jnp.float32),
    )(xf, ws, beta)


_NC, _NS, _L = 2, 16, 16
_NW = _NC * _NS
_RPW = (_B * _M) // _NW
_NBUF = 4


def _sc_gather_max_body(t2_hbm, idx_hbm, out_hbm, idx_v, bufs, sems, acc):
    wid = lax.axis_index("s") * _NC + lax.axis_index("c")
    base = wid * _RPW
    boff = (base // _M) * _N
    pltpu.sync_copy(idx_hbm.at[pl.ds(base * _K, _RPW * _K)], idx_v)

    def issue(m, j):
        iv = idx_v[pl.ds(m * _K, _K)] + boff
        pltpu.async_copy(t2_hbm.at[iv], bufs[j], sems[j])

    def wait(j):
        pltpu.make_async_copy(t2_hbm.at[pl.ds(0, _K)], bufs[j], sems[j]).wait()

    def compute(m, j):
        buf = bufs[j]
        for c in range(_C_OUT // _L):
            cs = pl.ds(c * _L, _L)
            mx = buf[0, cs]
            for k in range(1, _K):
                mx = jnp.maximum(mx, buf[k, cs])
            acc[m, cs] = jnp.maximum(mx, 0.0)

    for j in range(_NBUF):
        issue(j, j)

    def body(t, carry):
        for j in range(_NBUF):
            m = t * _NBUF + j
            wait(j)
            compute(m, j)

            @pl.when(m + _NBUF < _RPW)
            def _():
                issue(m + _NBUF, j)

        return carry

    lax.fori_loop(0, _RPW // _NBUF, body, 0)
    pltpu.sync_copy(acc, out_hbm.at[pl.ds(base, _RPW)])


def _sc_gather_max(t2_flat, idx_flat):
    mesh = plsc.VectorSubcoreMesh(core_axis_name="c", subcore_axis_name="s")
    f = functools.partial(
        pl.kernel,
        out_type=jax.ShapeDtypeStruct((_B * _M, _C_OUT), jnp.float32),
        mesh=mesh,
        scratch_types=[
            pltpu.VMEM((_RPW * _K,), jnp.int32),
            [pltpu.VMEM((_K, _C_OUT), jnp.float32) for _ in range(_NBUF)],
            [pltpu.SemaphoreType.DMA for _ in range(_NBUF)],
            pltpu.VMEM((_RPW, _C_OUT), jnp.float32),
        ],
    )(_sc_gather_max_body)
    return f(t2_flat, idx_flat)


def kernel(points_xyz, features, W, gamma, beta):
    xt = jnp.transpose(points_xyz, (2, 0, 1))
    nxyz = _fps(xt)
    new_xyz = jnp.transpose(nxyz, (1, 2, 0))

    idx = _knn(jnp.transpose(xt, (1, 0, 2)), jnp.transpose(nxyz, (1, 0, 2)))

    scale = gamma / jnp.sqrt(1.0 + _EPS)
    ws = (W * scale[:, None]).T
    xf = jnp.concatenate(
        [points_xyz, jnp.transpose(features, (0, 2, 1))], axis=-1)
    t2 = _t2(xf, ws, beta[None, :])
    pooled = _sc_gather_max(
        t2.reshape(_B * _N, _C_OUT), idx.reshape(_B * _M * _K))
    pooled = pooled.reshape(_B, _M, _C_OUT)
    return (new_xyz, jnp.transpose(pooled, (0, 2, 1)))
